# baseline (speedup 1.0000x reference)
import functools

import jax
import jax.numpy as jnp
from jax import lax
from jax.experimental import pallas as pl
from jax.experimental.pallas import tpu as pltpu
from jax.experimental.pallas import tpu_sc as plsc

NC = 2
NS = 16
LANES = 16
EB = 128


CH = 40
BPS = 160


def _chunk_sizes(total):
    out = []
    left = total
    while left > 0:
        c = min(CH, left)
        out.append(c)
        left -= c
    assert all(c % 8 == 0 and c % 2 == 0 for c in out)
    return out


def _make_sc_aggregate(d_feat, n_pad, nb):
    assert nb % BPS == 0 and nb // BPS == NS
    b0 = BPS
    rps = n_pad // NS
    zc = rps // EB
    assert rps % EB == 0

    mesh = plsc.VectorSubcoreMesh(core_axis_name="c", subcore_axis_name="s")
    out_type = jax.ShapeDtypeStruct((1, n_pad, d_feat), jnp.float32)

    scratch = [
        pltpu.VMEM((CH, EB), jnp.int32),
        pltpu.VMEM((CH, EB), jnp.int32),
        pltpu.VMEM((EB, d_feat), jnp.float32),
        pltpu.VMEM((EB, d_feat), jnp.float32),
        pltpu.VMEM_SHARED((n_pad, d_feat), jnp.float32),
        pltpu.SemaphoreType.DMA,
        pltpu.SemaphoreType.DMA,
    ]

    def body(feats, src2d, dst2d, parts, src_idx, dst_idx, buf0, buf1,
             accum, sem0, sem1):
        cid = lax.axis_index("c")
        sid = lax.axis_index("s")
        base = sid * rps

        @pl.when(cid == 0)
        def _():
            def zrow(i, carry):
                for c in range(d_feat // LANES):
                    buf0[i, pl.ds(c * LANES, LANES)] = jnp.zeros(
                        (LANES,), jnp.float32)
                return carry
            lax.fori_loop(0, EB, zrow, 0)
            for z in range(zc):
                pltpu.sync_copy(buf0, accum.at[pl.ds(base + z * EB, EB)])
        plsc.subcore_barrier()

        def gather(blk, buf, sem):
            pltpu.async_copy(feats.at[src_idx.at[blk]], buf, sem)

        def wait(buf, sem):
            pltpu.make_async_copy(feats.at[src_idx.at[0]], buf, sem).wait()

        def scat(buf, blk):
            pltpu.sync_copy(buf, accum.at[dst_idx.at[blk]], add=True)

        def pipe(blk0, csize):
            pltpu.sync_copy(src2d.at[pl.ds(blk0, csize)],
                            src_idx.at[pl.ds(0, csize)])
            pltpu.sync_copy(dst2d.at[pl.ds(blk0, csize)],
                            dst_idx.at[pl.ds(0, csize)])
            gather(0, buf0, sem0)

            def step(t, cc):
                b0_ = 2 * t
                b1_ = b0_ + 1
                gather(b1_, buf1, sem1)
                wait(buf0, sem0)
                scat(buf0, b0_)
                gather(lax.rem(b1_ + 1, csize), buf0, sem0)
                wait(buf1, sem1)
                scat(buf1, b1_)
                return cc
            lax.fori_loop(0, csize // 2, step, 0)
            wait(buf0, sem0)

        @pl.when(cid == 0)
        def _():
            pos = sid * b0
            for c in _chunk_sizes(b0):
                pipe(pos, c)
                pos = pos + c

        plsc.subcore_barrier()

        @pl.when(cid == 0)
        def _():
            pltpu.sync_copy(accum.at[pl.ds(base, rps)],
                            parts.at[0, pl.ds(base, rps)])

    return pl.kernel(body, out_type=out_type, mesh=mesh,
                     scratch_types=scratch)


def _make_sc_degree(n_pad, nb):
    bpw = nb // NS
    rps = n_pad // NS
    mesh = plsc.VectorSubcoreMesh(core_axis_name="c", subcore_axis_name="s")
    out_type = jax.ShapeDtypeStruct((NC, n_pad), jnp.float32)
    scratch = [
        pltpu.VMEM((bpw, EB), jnp.int32),
        pltpu.VMEM((n_pad,), jnp.float32),
        pltpu.VMEM((NS, rps), jnp.float32),
        pltpu.VMEM_SHARED((NS, NS, rps), jnp.float32),
    ]

    def body(dst2d, degp, dst_idx, hist, red, shist):
        cid = lax.axis_index("c")
        sid = lax.axis_index("s")
        base = sid * rps
        pltpu.sync_copy(dst2d.at[pl.ds(sid * bpw, bpw)], dst_idx)

        def zstep(i, c):
            hist[pl.ds(i * LANES, LANES)] = jnp.zeros((LANES,), jnp.float32)
            return c
        lax.fori_loop(0, n_pad // LANES, zstep, 0)
        ones16 = jnp.ones((LANES,), jnp.float32)

        def astep(t, c):
            b = t // (EB // LANES)
            l = t % (EB // LANES)
            idx = dst_idx[b, pl.ds(l * LANES, LANES)]
            plsc.addupdate_scatter(hist, [idx], ones16)
            return c
        lax.fori_loop(0, bpw * (EB // LANES), astep, 0)

        for r in range(NS):
            pltpu.sync_copy(hist.at[pl.ds(r * rps, rps)], shist.at[r, sid])
        plsc.subcore_barrier()
        pltpu.sync_copy(shist.at[sid], red)

        def rstep(v, c):
            acc = jnp.zeros((LANES,), jnp.float32)
            for hrow in range(NS):
                acc = acc + red[hrow, pl.ds(v * LANES, LANES)]
            hist[pl.ds(v * LANES, LANES)] = acc
            return c
        lax.fori_loop(0, rps // LANES, rstep, 0)
        pltpu.sync_copy(hist.at[pl.ds(0, rps)], degp.at[cid, pl.ds(base, rps)])

    return pl.kernel(body, out_type=out_type, mesh=mesh,
                     scratch_types=scratch,
                     compiler_params=pltpu.CompilerParams(
                         needs_layout_passes=False))


def _mm_body(x_ref, w_ref, o_ref):
    o_ref[...] = jnp.dot(x_ref[...], w_ref[...],
                         preferred_element_type=jnp.float32)


def _tc_matmul(x, w, bm=1024):
    n, d_in = x.shape
    d_out = w.shape[1]
    return pl.pallas_call(
        _mm_body,
        grid=(n // bm,),
        in_specs=[pl.BlockSpec((bm, d_in), lambda i: (i, 0)),
                  pl.BlockSpec((d_in, d_out), lambda i: (0, 0))],
        out_specs=pl.BlockSpec((bm, d_out), lambda i: (i, 0)),
        out_shape=jax.ShapeDtypeStruct((n, d_out), jnp.float32),
    )(x, w)


def _combine1_body(parts_ref, deg_ref, b1_ref, w2_ref, o_ref):
    a = parts_ref[0]
    r = jnp.transpose(1.0 / jnp.maximum(deg_ref[...], 1.0))
    h = jnp.maximum(a * r + b1_ref[...], 0.0)
    o_ref[...] = jnp.dot(h, w2_ref[...], preferred_element_type=jnp.float32)


def _tc_combine1(parts, degv, b1, w2, bm=1024):
    n = parts.shape[1]
    d = parts.shape[2]
    d2 = w2.shape[1]
    return pl.pallas_call(
        _combine1_body,
        grid=(n // bm,),
        in_specs=[pl.BlockSpec((1, bm, d), lambda i: (0, i, 0)),
                  pl.BlockSpec((1, bm), lambda i: (0, i)),
                  pl.BlockSpec((1, d), lambda i: (0, 0)),
                  pl.BlockSpec(w2.shape, lambda i: (0, 0))],
        out_specs=pl.BlockSpec((bm, d2), lambda i: (i, 0)),
        out_shape=jax.ShapeDtypeStruct((n, d2), jnp.float32),
    )(parts, degv, b1.reshape(1, -1), w2)


def _final_body(parts_ref, deg_ref, b2_ref, wc_ref, bc_ref, o_ref):
    a = parts_ref[0]
    r = jnp.transpose(1.0 / jnp.maximum(deg_ref[...], 1.0))
    h2 = a * r + b2_ref[...]
    z = jnp.dot(h2, wc_ref[...], preferred_element_type=jnp.float32) \
        + bc_ref[...]
    m = jnp.max(z, axis=-1, keepdims=True)
    e = jnp.exp(z - m)
    o_ref[...] = (z - m) - jnp.log(jnp.sum(e, axis=-1, keepdims=True))


def _tc_final(parts, degv, b2, wc, bc, bm=1024):
    n = parts.shape[1]
    dh = parts.shape[2]
    ncls = wc.shape[1]
    return pl.pallas_call(
        _final_body,
        grid=(n // bm,),
        in_specs=[pl.BlockSpec((1, bm, dh), lambda i: (0, i, 0)),
                  pl.BlockSpec((1, bm), lambda i: (0, i)),
                  pl.BlockSpec((1, dh), lambda i: (0, 0)),
                  pl.BlockSpec(wc.shape, lambda i: (0, 0)),
                  pl.BlockSpec((1, ncls), lambda i: (0, 0))],
        out_specs=pl.BlockSpec((bm, ncls), lambda i: (i, 0)),
        out_shape=jax.ShapeDtypeStruct((n, ncls), jnp.float32),
    )(parts, degv, b2.reshape(1, -1), wc, bc.reshape(1, -1))


def kernel(x, edge_index, W1, b1, W2, b2, Wc, bc):
    n, d_in = x.shape
    e = edge_index.shape[1]

    n_pad = -(-n // (NS * EB)) * (NS * EB)
    if n_pad == n:
        n_pad += NS * EB
    blk_unit = NS * BPS * EB
    e_pad = -(-e // blk_unit) * blk_unit
    nb = e_pad // EB

    xp = jnp.zeros((n_pad, d_in), x.dtype).at[:n].set(x)
    pad = e_pad - e
    src = jnp.concatenate(
        [edge_index[0], jnp.zeros((pad,), jnp.int32)]).reshape(nb, EB)
    dst = jnp.concatenate(
        [edge_index[1], jnp.full((pad,), n_pad - 1, jnp.int32)]).reshape(nb, EB)

    y1 = _tc_matmul(xp, W1)
    degp = _make_sc_degree(n_pad, nb)(dst)
    degv = degp[:1]
    parts1 = _make_sc_aggregate(W1.shape[1], n_pad, nb)(y1, src, dst)
    y2 = _tc_combine1(parts1, degv, b1, W2)
    parts2 = _make_sc_aggregate(W2.shape[1], n_pad, nb)(y2, src, dst)
    out = _tc_final(parts2, degv, b2, Wc, bc)
    return out[:n]


# speedup vs baseline: 3.3254x; 3.3254x over previous
import functools

import jax
import jax.numpy as jnp
from jax import lax
from jax.experimental import pallas as pl
from jax.experimental.pallas import tpu as pltpu
from jax.experimental.pallas import tpu_sc as plsc

NC = 2
NS = 16
LANES = 16
EB = 128


CH = 40
BPS = 160


def _chunk_sizes(total):
    out = []
    left = total
    while left > 0:
        c = min(CH, left)
        out.append(c)
        left -= c
    assert all(c % 8 == 0 and c % 2 == 0 for c in out)
    return out


def _make_sc_aggregate(d_feat, n_pad, nb):
    assert nb % BPS == 0 and nb // BPS == NS
    bpw = BPS // NC
    rps = n_pad // NS
    zc = rps // EB
    assert rps % EB == 0

    mesh = plsc.VectorSubcoreMesh(core_axis_name="c", subcore_axis_name="s")
    out_type = jax.ShapeDtypeStruct((NC, n_pad, d_feat), jnp.float32)

    scratch = [
        pltpu.VMEM((CH, EB), jnp.int32),
        pltpu.VMEM((CH, EB), jnp.int32),
        pltpu.VMEM((EB, d_feat), jnp.float32),
        pltpu.VMEM((EB, d_feat), jnp.float32),
        pltpu.VMEM_SHARED((n_pad, d_feat), jnp.float32),
        pltpu.SemaphoreType.DMA,
        pltpu.SemaphoreType.DMA,
    ]

    def body(feats, src2d, dst2d, parts, src_idx, dst_idx, buf0, buf1,
             accum, sem0, sem1):
        cid = lax.axis_index("c")
        sid = lax.axis_index("s")
        base = sid * rps

        def zrow(i, carry):
            for c in range(d_feat // LANES):
                buf0[i, pl.ds(c * LANES, LANES)] = jnp.zeros(
                    (LANES,), jnp.float32)
            return carry
        lax.fori_loop(0, EB, zrow, 0)
        for z in range(zc):
            pltpu.sync_copy(buf0, accum.at[pl.ds(base + z * EB, EB)])
        plsc.subcore_barrier()

        def gather(blk, buf, sem):
            pltpu.async_copy(feats.at[src_idx.at[blk]], buf, sem)

        def wait(buf, sem):
            pltpu.make_async_copy(feats.at[src_idx.at[0]], buf, sem).wait()

        def scat(buf, blk):
            pltpu.sync_copy(buf, accum.at[dst_idx.at[blk]], add=True)

        def pipe(blk0, csize):
            pltpu.sync_copy(src2d.at[pl.ds(blk0, csize)],
                            src_idx.at[pl.ds(0, csize)])
            pltpu.sync_copy(dst2d.at[pl.ds(blk0, csize)],
                            dst_idx.at[pl.ds(0, csize)])
            gather(0, buf0, sem0)

            def step(t, cc):
                b0_ = 2 * t
                b1_ = b0_ + 1
                gather(b1_, buf1, sem1)
                wait(buf0, sem0)
                scat(buf0, b0_)
                gather(lax.rem(b1_ + 1, csize), buf0, sem0)
                wait(buf1, sem1)
                scat(buf1, b1_)
                return cc
            lax.fori_loop(0, csize // 2, step, 0)
            wait(buf0, sem0)

        wid = sid * NC + cid
        pos = wid * bpw
        for c in _chunk_sizes(bpw):
            pipe(pos, c)
            pos = pos + c

        plsc.subcore_barrier()

        pltpu.sync_copy(accum.at[pl.ds(base, rps)],
                        parts.at[cid, pl.ds(base, rps)])

    return pl.kernel(body, out_type=out_type, mesh=mesh,
                     scratch_types=scratch)


def _make_sc_degree(n_pad, nb):
    bpw = nb // NS
    rps = n_pad // NS
    mesh = plsc.VectorSubcoreMesh(core_axis_name="c", subcore_axis_name="s")
    out_type = jax.ShapeDtypeStruct((NC, n_pad), jnp.float32)
    scratch = [
        pltpu.VMEM((bpw, EB), jnp.int32),
        pltpu.VMEM((n_pad,), jnp.float32),
        pltpu.VMEM((NS, rps), jnp.float32),
        pltpu.VMEM_SHARED((NS, NS, rps), jnp.float32),
    ]

    def body(dst2d, degp, dst_idx, hist, red, shist):
        cid = lax.axis_index("c")
        sid = lax.axis_index("s")
        base = sid * rps
        pltpu.sync_copy(dst2d.at[pl.ds(sid * bpw, bpw)], dst_idx)

        def zstep(i, c):
            hist[pl.ds(i * LANES, LANES)] = jnp.zeros((LANES,), jnp.float32)
            return c
        lax.fori_loop(0, n_pad // LANES, zstep, 0)
        ones16 = jnp.ones((LANES,), jnp.float32)

        def astep(t, c):
            b = t // (EB // LANES)
            l = t % (EB // LANES)
            idx = dst_idx[b, pl.ds(l * LANES, LANES)]
            plsc.addupdate_scatter(hist, [idx], ones16)
            return c
        lax.fori_loop(0, bpw * (EB // LANES), astep, 0)

        for r in range(NS):
            pltpu.sync_copy(hist.at[pl.ds(r * rps, rps)], shist.at[r, sid])
        plsc.subcore_barrier()
        pltpu.sync_copy(shist.at[sid], red)

        def rstep(v, c):
            acc = jnp.zeros((LANES,), jnp.float32)
            for hrow in range(NS):
                acc = acc + red[hrow, pl.ds(v * LANES, LANES)]
            hist[pl.ds(v * LANES, LANES)] = acc
            return c
        lax.fori_loop(0, rps // LANES, rstep, 0)
        pltpu.sync_copy(hist.at[pl.ds(0, rps)], degp.at[cid, pl.ds(base, rps)])

    return pl.kernel(body, out_type=out_type, mesh=mesh,
                     scratch_types=scratch,
                     compiler_params=pltpu.CompilerParams(
                         needs_layout_passes=False))


def _mm_body(x_ref, w_ref, o_ref):
    o_ref[...] = jnp.dot(x_ref[...], w_ref[...],
                         preferred_element_type=jnp.float32)


def _tc_matmul(x, w, bm=1024):
    n, d_in = x.shape
    d_out = w.shape[1]
    return pl.pallas_call(
        _mm_body,
        grid=(n // bm,),
        in_specs=[pl.BlockSpec((bm, d_in), lambda i: (i, 0)),
                  pl.BlockSpec((d_in, d_out), lambda i: (0, 0))],
        out_specs=pl.BlockSpec((bm, d_out), lambda i: (i, 0)),
        out_shape=jax.ShapeDtypeStruct((n, d_out), jnp.float32),
    )(x, w)


def _combine1_body(parts_ref, deg_ref, b1_ref, w2_ref, o_ref):
    a = parts_ref[0] + parts_ref[1]
    r = jnp.transpose(1.0 / jnp.maximum(deg_ref[...], 1.0))
    h = jnp.maximum(a * r + b1_ref[...], 0.0)
    o_ref[...] = jnp.dot(h, w2_ref[...], preferred_element_type=jnp.float32)


def _tc_combine1(parts, degv, b1, w2, bm=1024):
    n = parts.shape[1]
    d = parts.shape[2]
    d2 = w2.shape[1]
    return pl.pallas_call(
        _combine1_body,
        grid=(n // bm,),
        in_specs=[pl.BlockSpec((NC, bm, d), lambda i: (0, i, 0)),
                  pl.BlockSpec((1, bm), lambda i: (0, i)),
                  pl.BlockSpec((1, d), lambda i: (0, 0)),
                  pl.BlockSpec(w2.shape, lambda i: (0, 0))],
        out_specs=pl.BlockSpec((bm, d2), lambda i: (i, 0)),
        out_shape=jax.ShapeDtypeStruct((n, d2), jnp.float32),
    )(parts, degv, b1.reshape(1, -1), w2)


def _final_body(parts_ref, deg_ref, b2_ref, wc_ref, bc_ref, o_ref):
    a = parts_ref[0] + parts_ref[1]
    r = jnp.transpose(1.0 / jnp.maximum(deg_ref[...], 1.0))
    h2 = a * r + b2_ref[...]
    z = jnp.dot(h2, wc_ref[...], preferred_element_type=jnp.float32) \
        + bc_ref[...]
    m = jnp.max(z, axis=-1, keepdims=True)
    e = jnp.exp(z - m)
    o_ref[...] = (z - m) - jnp.log(jnp.sum(e, axis=-1, keepdims=True))


def _tc_final(parts, degv, b2, wc, bc, bm=1024):
    n = parts.shape[1]
    dh = parts.shape[2]
    ncls = wc.shape[1]
    return pl.pallas_call(
        _final_body,
        grid=(n // bm,),
        in_specs=[pl.BlockSpec((NC, bm, dh), lambda i: (0, i, 0)),
                  pl.BlockSpec((1, bm), lambda i: (0, i)),
                  pl.BlockSpec((1, dh), lambda i: (0, 0)),
                  pl.BlockSpec(wc.shape, lambda i: (0, 0)),
                  pl.BlockSpec((1, ncls), lambda i: (0, 0))],
        out_specs=pl.BlockSpec((bm, ncls), lambda i: (i, 0)),
        out_shape=jax.ShapeDtypeStruct((n, ncls), jnp.float32),
    )(parts, degv, b2.reshape(1, -1), wc, bc.reshape(1, -1))


def kernel(x, edge_index, W1, b1, W2, b2, Wc, bc):
    n, d_in = x.shape
    e = edge_index.shape[1]

    n_pad = -(-n // (NS * EB)) * (NS * EB)
    if n_pad - n < EB:
        n_pad += NS * EB
    blk_unit = NS * BPS * EB
    e_pad = -(-e // blk_unit) * blk_unit
    nb = e_pad // EB

    xp = jnp.zeros((n_pad, d_in), x.dtype).at[:n].set(x)
    pad = e_pad - e
    pad_src = (jnp.arange(pad, dtype=jnp.int32) % n)
    pad_dst = n + (jnp.arange(pad, dtype=jnp.int32) % (n_pad - n))
    src = jnp.concatenate([edge_index[0], pad_src]).reshape(nb, EB)
    dst = jnp.concatenate([edge_index[1], pad_dst]).reshape(nb, EB)

    y1 = _tc_matmul(xp, W1)
    degp = _make_sc_degree(n_pad, nb)(dst)
    degv = degp[:1]
    parts1 = _make_sc_aggregate(W1.shape[1], n_pad, nb)(y1, src, dst)
    y2 = _tc_combine1(parts1, degv, b1, W2)
    parts2 = _make_sc_aggregate(W2.shape[1], n_pad, nb)(y2, src, dst)
    out = _tc_final(parts2, degv, b2, Wc, bc)
    return out[:n]
